# Initial kernel scaffold; baseline (speedup 1.0000x reference)
import jax
import jax.numpy as jnp
from jax.experimental import pallas as pl


def kernel(f_atoms, f_bonds, edge_src, edge_dst, W_i_atom, W_i_bond, W_h_0, W_h_1, W_lr, W_o, b_o, gru_Wih_f, gru_Whh_f, gru_bih_f, gru_bhh_f, gru_Wih_b, gru_Whh_b, gru_bih_b, gru_bhh_b):
    raise NotImplementedError("write your pallas kernel here")


# speedup vs baseline: 2.0424x; 2.0424x over previous
import functools

import jax
import jax.numpy as jnp
from jax import lax
from jax.experimental import pallas as pl
from jax.experimental.pallas import tpu as pltpu
from jax.experimental.pallas import tpu_sc as plsc

H = 128
NC, NS, L = 2, 16, 16
NW = NC * NS
EDGE_BLK = 6400
CHUNK = 128
FLUSH = 128
IDS_CAP = 2 * FLUSH + 16


def _dot16(a, w):
    return jnp.dot(a.astype(jnp.bfloat16), w.astype(jnp.bfloat16),
                   preferred_element_type=jnp.float32)


def _mm_relu_kernel(a_ref, w_ref, o_ref):
    o_ref[...] = jnp.maximum(_dot16(a_ref[...], w_ref[...]), 0.0)


def _mm_relu(a, w_t, blk):
    m, k = a.shape
    n = w_t.shape[1]
    return pl.pallas_call(
        _mm_relu_kernel,
        grid=(m // blk,),
        in_specs=[pl.BlockSpec((blk, k), lambda i: (i, 0)),
                  pl.BlockSpec((k, n), lambda i: (0, 0))],
        out_specs=pl.BlockSpec((blk, n), lambda i: (i, 0)),
        out_shape=jax.ShapeDtypeStruct((m, n), jnp.float32),
    )(a, w_t)


def _seg_body(npt, n_edges, x_hbm, edst_hbm, out_hbm,
              edst_v, ids_v, dsts_v, fids_v, fdst_v, rows_v,
              accs_v, accm_v, sem):
    wid = lax.axis_index("s") * NC + lax.axis_index("c")
    lo = wid * npt

    zf = jnp.zeros((L,), jnp.float32)
    zi = jnp.zeros((L,), jnp.int32)

    def zero_acc(i, _):
        accs_v[pl.ds(i * L, L)] = zf
        accm_v[pl.ds(i * L, L)] = zf
        return 0
    lax.fori_loop(0, (npt + 1) * (H // L), zero_acc, 0)
    for k in range(IDS_CAP // L):
        ids_v[pl.ds(k * L, L)] = zi
        dsts_v[pl.ds(k * L, L)] = zi

    def do_flush(cnt):
        nb = jnp.minimum(cnt, FLUSH)
        for k in range(FLUSH // L):
            fids_v[pl.ds(k * L, L)] = ids_v[pl.ds(k * L, L)]
            fdst_v[pl.ds(k * L, L)] = dsts_v[pl.ds(k * L, L)]
        pltpu.async_copy(x_hbm.at[fids_v], rows_v, sem).wait()

        def acc_grp(jg, _):
            dvec = fdst_v[pl.ds(jg * L, L)]
            idxv = lax.iota(jnp.int32, L) + jg * L
            dvec = jnp.where(idxv < nb, dvec, npt)
            for k16 in range(L):
                off = dvec[k16] * H
                j = jg * L + k16
                for k in range(H // L):
                    r = rows_v[j, pl.ds(k * L, L)]
                    plsc.addupdate(accs_v.at[pl.ds(off + k * L, L)], r)
                    cur = accm_v[pl.ds(off + k * L, L)]
                    accm_v[pl.ds(off + k * L, L)] = jnp.maximum(cur, r)
            return 0
        lax.fori_loop(0, FLUSH // L, acc_grp, 0)
        for k in range(FLUSH // L):
            ids_v[pl.ds(k * L, L)] = ids_v[pl.ds(FLUSH + k * L, L)]
            dsts_v[pl.ds(k * L, L)] = dsts_v[pl.ds(FLUSH + k * L, L)]
        return cnt - nb

    def blk_body(b, cnt):
        pltpu.sync_copy(edst_hbm.at[pl.ds(b * EDGE_BLK, EDGE_BLK)], edst_v)

        def chunk_body(c, cnt):
            base = b * EDGE_BLK + c * CHUNK
            for u in range(CHUNK // L):
                dst = edst_v[pl.ds(c * CHUNK + u * L, L)]
                eid = lax.iota(jnp.int32, L) + (base + u * L)
                msk = (dst >= lo) & (dst < lo + npt)
                plsc.store_compressed(ids_v.at[pl.ds(cnt, L)], eid, mask=msk)
                plsc.store_compressed(dsts_v.at[pl.ds(cnt, L)], dst - lo, mask=msk)
                cnt = cnt + jnp.max(plsc.all_reduce_population_count(msk))
            return lax.cond(cnt >= FLUSH, do_flush, lambda c_: c_, cnt)
        return lax.fori_loop(0, EDGE_BLK // CHUNK, chunk_body, cnt)

    cnt = lax.fori_loop(0, n_edges // EDGE_BLK, blk_body, jnp.int32(0))
    cnt = lax.cond(cnt > 0, do_flush, lambda c_: c_, cnt)

    def prod_row(i, _):
        accs_v[pl.ds(i * L, L)] = accs_v[pl.ds(i * L, L)] * accm_v[pl.ds(i * L, L)]
        return 0
    lax.fori_loop(0, npt * (H // L), prod_row, 0)
    pltpu.sync_copy(accs_v.at[pl.ds(0, npt * H)], out_hbm.at[pl.ds(lo * H, npt * H)])


def _segment_summax(x, edge_dst, n_pad):
    npt = n_pad // NW
    n_edges = x.shape[0]
    mesh = plsc.VectorSubcoreMesh(core_axis_name="c", subcore_axis_name="s",
                                  num_cores=NC, num_subcores=NS)
    body = functools.partial(_seg_body, npt, n_edges)
    out = pl.kernel(
        body,
        out_type=jax.ShapeDtypeStruct((n_pad * H,), jnp.float32),
        mesh=mesh,
        scratch_types=[
            pltpu.VMEM((EDGE_BLK,), jnp.int32),
            pltpu.VMEM((IDS_CAP,), jnp.int32),
            pltpu.VMEM((IDS_CAP,), jnp.int32),
            pltpu.VMEM((FLUSH,), jnp.int32),
            pltpu.VMEM((FLUSH,), jnp.int32),
            pltpu.VMEM((FLUSH, H), jnp.float32),
            pltpu.VMEM(((npt + 1) * H,), jnp.float32),
            pltpu.VMEM(((npt + 1) * H,), jnp.float32),
            pltpu.SemaphoreType.DMA,
        ],
        compiler_params=pltpu.CompilerParams(needs_layout_passes=False),
    )(x, edge_dst)
    return out.reshape(n_pad, H)


def _gates_kernel(mg_ref, ia_ref, w1_ref, w2_ref, w3_ref, wif_ref, bif_ref,
                  wib_ref, bib_ref, h0_ref, gf_ref, gb_ref):
    mg = mg_ref[...]
    ia = ia_ref[...]
    f = ia + 2.0 * mg
    am = ((_dot16(mg, w1_ref[...]) + _dot16(f, w2_ref[...]))
          + _dot16(ia, w3_ref[...]))
    t = am.shape[0]
    h0_ref[...] = jnp.max(am, axis=0).reshape(1, 1, H)
    gf = _dot16(am, wif_ref[...]) + bif_ref[...]
    gb = _dot16(am, wib_ref[...]) + bib_ref[...]
    gf_ref[...] = gf.reshape(t, 1, 1, 3 * H)
    gb_ref[...] = gb.reshape(t, 1, 1, 3 * H)


def _gru_kernel(n_per, gf_ref, gb_ref, h0_ref, whf_ref, bhf_ref,
                whb_ref, bhb_ref, wof_ref, wob_ref, bo_ref, o_ref):
    b = h0_ref.shape[0]
    h0 = h0_ref[...]
    zero = jnp.zeros((b, H), jnp.float32)

    def gru_step(g, h, wh_ref, bh_ref):
        gh = _dot16(h, wh_ref[...]) + bh_ref[...]
        r = jax.nn.sigmoid(g[:, :H] + gh[:, :H])
        z = jax.nn.sigmoid(g[:, H:2 * H] + gh[:, H:2 * H])
        n = jnp.tanh(g[:, 2 * H:] + r * gh[:, 2 * H:])
        return (1.0 - z) * n + z * h

    def step(t, carry):
        hf, hb, sf, sb = carry
        hf = gru_step(gf_ref[t], hf, whf_ref, bhf_ref)
        hb = gru_step(gb_ref[n_per - 1 - t], hb, whb_ref, bhb_ref)
        return (hf, hb, sf + hf, sb + hb)

    _, _, sf, sb = lax.fori_loop(0, n_per, step, (h0, h0, zero, zero))
    inv = 1.0 / n_per
    emb = (_dot16(sf * inv, wof_ref[...]) + _dot16(sb * inv, wob_ref[...])
           + bo_ref[...])
    o_ref[...] = jnp.maximum(emb, 0.0)


def kernel(f_atoms, f_bonds, edge_src, edge_dst, W_i_atom, W_i_bond, W_h_0, W_h_1,
           W_lr, W_o, b_o, gru_Wih_f, gru_Whh_f, gru_bih_f, gru_bhh_f,
           gru_Wih_b, gru_Whh_b, gru_bih_b, gru_bhh_b):
    n_nodes = f_atoms.shape[0]
    n_graphs = 50
    n_per = n_nodes // n_graphs
    n_pad = ((n_nodes + NW - 1) // NW) * NW

    ia = _mm_relu(f_atoms, W_i_atom.T, 1000)
    x = _mm_relu(f_bonds, W_i_bond.T, 512)

    messge = _segment_summax(x, edge_dst.astype(jnp.int32), n_pad)

    h0, gf, gb = pl.pallas_call(
        _gates_kernel,
        grid=(n_graphs,),
        in_specs=[
            pl.BlockSpec((n_per, H), lambda i: (i, 0)),
            pl.BlockSpec((n_per, H), lambda i: (i, 0)),
            pl.BlockSpec((H, H), lambda i: (0, 0)),
            pl.BlockSpec((H, H), lambda i: (0, 0)),
            pl.BlockSpec((H, H), lambda i: (0, 0)),
            pl.BlockSpec((H, 3 * H), lambda i: (0, 0)),
            pl.BlockSpec((1, 3 * H), lambda i: (0, 0)),
            pl.BlockSpec((H, 3 * H), lambda i: (0, 0)),
            pl.BlockSpec((1, 3 * H), lambda i: (0, 0)),
        ],
        out_specs=[
            pl.BlockSpec((1, 1, H), lambda i: (i, 0, 0)),
            pl.BlockSpec((n_per, 1, 1, 3 * H), lambda i: (0, i, 0, 0)),
            pl.BlockSpec((n_per, 1, 1, 3 * H), lambda i: (0, i, 0, 0)),
        ],
        out_shape=[
            jax.ShapeDtypeStruct((n_graphs, 1, H), jnp.float32),
            jax.ShapeDtypeStruct((n_per, n_graphs, 1, 3 * H), jnp.float32),
            jax.ShapeDtypeStruct((n_per, n_graphs, 1, 3 * H), jnp.float32),
        ],
    )(messge, ia, W_lr[:, :H].T, W_lr[:, H:2 * H].T, W_lr[:, 2 * H:].T,
      gru_Wih_f.T, gru_bih_f.reshape(1, 3 * H),
      gru_Wih_b.T, gru_bih_b.reshape(1, 3 * H))

    out = pl.pallas_call(
        functools.partial(_gru_kernel, n_per),
        out_shape=jax.ShapeDtypeStruct((n_graphs, H), jnp.float32),
    )(gf.reshape(n_per, n_graphs, 3 * H), gb.reshape(n_per, n_graphs, 3 * H),
      h0.reshape(n_graphs, H), gru_Whh_f.T, gru_bhh_f.reshape(1, 3 * H),
      gru_Whh_b.T, gru_bhh_b.reshape(1, 3 * H),
      W_o[:, :H].T, W_o[:, H:].T, b_o.reshape(1, H))
    return out


# speedup vs baseline: 2.5658x; 1.2563x over previous
import functools

import jax
import jax.numpy as jnp
from jax import lax
from jax.experimental import pallas as pl
from jax.experimental.pallas import tpu as pltpu
from jax.experimental.pallas import tpu_sc as plsc

H = 128
NC, NS, L = 2, 16, 16
NW = NC * NS
EDGE_BLK = 6400
CHUNK = 128
FLUSH = 128
IDS_CAP = 2 * FLUSH + 16


def _dot16(a, w):
    return jnp.dot(a.astype(jnp.bfloat16), w.astype(jnp.bfloat16),
                   preferred_element_type=jnp.float32)


def _mm_relu_kernel(a_ref, w_ref, o_ref):
    o_ref[...] = jnp.maximum(_dot16(a_ref[...], w_ref[...]), 0.0)


def _mm_relu(a, w_t, blk):
    m, k = a.shape
    n = w_t.shape[1]
    return pl.pallas_call(
        _mm_relu_kernel,
        grid=(m // blk,),
        in_specs=[pl.BlockSpec((blk, k), lambda i: (i, 0)),
                  pl.BlockSpec((k, n), lambda i: (0, 0))],
        out_specs=pl.BlockSpec((blk, n), lambda i: (i, 0)),
        out_shape=jax.ShapeDtypeStruct((m, n), jnp.float32),
    )(a, w_t)


def _seg_body(npt, n_edges, x_hbm, edst_hbm, out_hbm,
              edst_v, ids_v, dsts_v, fids_v, fdst_v, rows_v,
              accs_v, accm_v, sem):
    wid = lax.axis_index("s") * NC + lax.axis_index("c")
    lo = wid * npt

    zf = jnp.zeros((L,), jnp.float32)
    zi = jnp.zeros((L,), jnp.int32)

    def zero_acc(i, _):
        accs_v[pl.ds(i * L, L)] = zf
        accm_v[pl.ds(i * L, L)] = zf
        return 0
    lax.fori_loop(0, (npt + 1) * (H // L), zero_acc, 0)
    for k in range(IDS_CAP // L):
        ids_v[pl.ds(k * L, L)] = zi
        dsts_v[pl.ds(k * L, L)] = zi

    def do_flush(cnt):
        nb = jnp.minimum(cnt, FLUSH)
        for k in range(FLUSH // L):
            fids_v[pl.ds(k * L, L)] = ids_v[pl.ds(k * L, L)]
            fdst_v[pl.ds(k * L, L)] = dsts_v[pl.ds(k * L, L)]
        pltpu.async_copy(x_hbm.at[fids_v], rows_v, sem).wait()

        def acc_grp(jg, _):
            dvec = fdst_v[pl.ds(jg * L, L)]
            idxv = lax.iota(jnp.int32, L) + jg * L
            dvec = jnp.where(idxv < nb, dvec, npt)
            for k16 in range(L):
                off = dvec[k16] * H
                j = jg * L + k16
                for k in range(H // L):
                    r = rows_v[j, pl.ds(k * L, L)]
                    plsc.addupdate(accs_v.at[pl.ds(off + k * L, L)], r)
                    cur = accm_v[pl.ds(off + k * L, L)]
                    accm_v[pl.ds(off + k * L, L)] = jnp.maximum(cur, r)
            return 0
        lax.fori_loop(0, FLUSH // L, acc_grp, 0)
        for k in range(FLUSH // L):
            ids_v[pl.ds(k * L, L)] = ids_v[pl.ds(FLUSH + k * L, L)]
            dsts_v[pl.ds(k * L, L)] = dsts_v[pl.ds(FLUSH + k * L, L)]
        return cnt - nb

    def blk_body(b, cnt):
        pltpu.sync_copy(edst_hbm.at[pl.ds(b * EDGE_BLK, EDGE_BLK)], edst_v)

        def chunk_body(c, cnt):
            base = b * EDGE_BLK + c * CHUNK
            for u in range(CHUNK // L):
                dst = edst_v[pl.ds(c * CHUNK + u * L, L)]
                eid = lax.iota(jnp.int32, L) + (base + u * L)
                msk = (dst >= lo) & (dst < lo + npt)
                plsc.store_compressed(ids_v.at[pl.ds(cnt, L)], eid, mask=msk)
                plsc.store_compressed(dsts_v.at[pl.ds(cnt, L)], dst - lo, mask=msk)
                cnt = cnt + plsc.all_reduce_population_count(msk)[0]
            return lax.cond(cnt >= FLUSH, do_flush, lambda c_: c_, cnt)
        return lax.fori_loop(0, EDGE_BLK // CHUNK, chunk_body, cnt)

    cnt = lax.fori_loop(0, n_edges // EDGE_BLK, blk_body, jnp.int32(0))
    cnt = lax.cond(cnt > 0, do_flush, lambda c_: c_, cnt)

    def prod_row(i, _):
        accs_v[pl.ds(i * L, L)] = accs_v[pl.ds(i * L, L)] * accm_v[pl.ds(i * L, L)]
        return 0
    lax.fori_loop(0, npt * (H // L), prod_row, 0)
    pltpu.sync_copy(accs_v.at[pl.ds(0, npt * H)], out_hbm.at[pl.ds(lo * H, npt * H)])


def _segment_summax(x, edge_dst, n_pad):
    npt = n_pad // NW
    n_edges = x.shape[0]
    mesh = plsc.VectorSubcoreMesh(core_axis_name="c", subcore_axis_name="s",
                                  num_cores=NC, num_subcores=NS)
    body = functools.partial(_seg_body, npt, n_edges)
    out = pl.kernel(
        body,
        out_type=jax.ShapeDtypeStruct((n_pad * H,), jnp.float32),
        mesh=mesh,
        scratch_types=[
            pltpu.VMEM((EDGE_BLK,), jnp.int32),
            pltpu.VMEM((IDS_CAP,), jnp.int32),
            pltpu.VMEM((IDS_CAP,), jnp.int32),
            pltpu.VMEM((FLUSH,), jnp.int32),
            pltpu.VMEM((FLUSH,), jnp.int32),
            pltpu.VMEM((FLUSH, H), jnp.float32),
            pltpu.VMEM(((npt + 1) * H,), jnp.float32),
            pltpu.VMEM(((npt + 1) * H,), jnp.float32),
            pltpu.SemaphoreType.DMA,
        ],
        compiler_params=pltpu.CompilerParams(needs_layout_passes=False),
    )(x, edge_dst)
    return out.reshape(n_pad, H)


def _gates_kernel(mg_ref, ia_ref, w1_ref, w2_ref, w3_ref, wif_ref, bif_ref,
                  wib_ref, bib_ref, h0_ref, gf_ref, gb_ref):
    mg = mg_ref[...]
    ia = ia_ref[...]
    f = ia + 2.0 * mg
    am = ((_dot16(mg, w1_ref[...]) + _dot16(f, w2_ref[...]))
          + _dot16(ia, w3_ref[...]))
    t = am.shape[0]
    h0_ref[...] = jnp.max(am, axis=0).reshape(1, 1, H)
    gf = _dot16(am, wif_ref[...]) + bif_ref[...]
    gb = _dot16(am, wib_ref[...]) + bib_ref[...]
    gf_ref[...] = gf.reshape(t, 1, 1, 3 * H)
    gb_ref[...] = gb.reshape(t, 1, 1, 3 * H)


def _gru_kernel(n_per, gf_ref, gb_ref, h0_ref, whf_ref, bhf_ref,
                whb_ref, bhb_ref, wof_ref, wob_ref, bo_ref, o_ref):
    b = h0_ref.shape[0]
    h0 = h0_ref[...]
    zero = jnp.zeros((b, H), jnp.float32)

    def gru_step(g, h, wh_ref, bh_ref):
        gh = _dot16(h, wh_ref[...]) + bh_ref[...]
        r = jax.nn.sigmoid(g[:, :H] + gh[:, :H])
        z = jax.nn.sigmoid(g[:, H:2 * H] + gh[:, H:2 * H])
        n = jnp.tanh(g[:, 2 * H:] + r * gh[:, 2 * H:])
        return (1.0 - z) * n + z * h

    def step(t, carry):
        hf, hb, sf, sb = carry
        hf = gru_step(gf_ref[t], hf, whf_ref, bhf_ref)
        hb = gru_step(gb_ref[n_per - 1 - t], hb, whb_ref, bhb_ref)
        return (hf, hb, sf + hf, sb + hb)

    _, _, sf, sb = lax.fori_loop(0, n_per, step, (h0, h0, zero, zero))
    inv = 1.0 / n_per
    emb = (_dot16(sf * inv, wof_ref[...]) + _dot16(sb * inv, wob_ref[...])
           + bo_ref[...])
    o_ref[...] = jnp.maximum(emb, 0.0)


def kernel(f_atoms, f_bonds, edge_src, edge_dst, W_i_atom, W_i_bond, W_h_0, W_h_1,
           W_lr, W_o, b_o, gru_Wih_f, gru_Whh_f, gru_bih_f, gru_bhh_f,
           gru_Wih_b, gru_Whh_b, gru_bih_b, gru_bhh_b):
    n_nodes = f_atoms.shape[0]
    n_graphs = 50
    n_per = n_nodes // n_graphs
    n_pad = ((n_nodes + NW - 1) // NW) * NW

    ia = _mm_relu(f_atoms, W_i_atom.T, 1000)
    x = _mm_relu(f_bonds, W_i_bond.T, 3200)

    messge = _segment_summax(x, edge_dst.astype(jnp.int32), n_pad)

    h0, gf, gb = pl.pallas_call(
        _gates_kernel,
        grid=(n_graphs,),
        in_specs=[
            pl.BlockSpec((n_per, H), lambda i: (i, 0)),
            pl.BlockSpec((n_per, H), lambda i: (i, 0)),
            pl.BlockSpec((H, H), lambda i: (0, 0)),
            pl.BlockSpec((H, H), lambda i: (0, 0)),
            pl.BlockSpec((H, H), lambda i: (0, 0)),
            pl.BlockSpec((H, 3 * H), lambda i: (0, 0)),
            pl.BlockSpec((1, 3 * H), lambda i: (0, 0)),
            pl.BlockSpec((H, 3 * H), lambda i: (0, 0)),
            pl.BlockSpec((1, 3 * H), lambda i: (0, 0)),
        ],
        out_specs=[
            pl.BlockSpec((1, 1, H), lambda i: (i, 0, 0)),
            pl.BlockSpec((n_per, 1, 1, 3 * H), lambda i: (0, i, 0, 0)),
            pl.BlockSpec((n_per, 1, 1, 3 * H), lambda i: (0, i, 0, 0)),
        ],
        out_shape=[
            jax.ShapeDtypeStruct((n_graphs, 1, H), jnp.float32),
            jax.ShapeDtypeStruct((n_per, n_graphs, 1, 3 * H), jnp.float32),
            jax.ShapeDtypeStruct((n_per, n_graphs, 1, 3 * H), jnp.float32),
        ],
    )(messge, ia, W_lr[:, :H].T, W_lr[:, H:2 * H].T, W_lr[:, 2 * H:].T,
      gru_Wih_f.T, gru_bih_f.reshape(1, 3 * H),
      gru_Wih_b.T, gru_bih_b.reshape(1, 3 * H))

    out = pl.pallas_call(
        functools.partial(_gru_kernel, n_per),
        out_shape=jax.ShapeDtypeStruct((n_graphs, H), jnp.float32),
    )(gf.reshape(n_per, n_graphs, 3 * H), gb.reshape(n_per, n_graphs, 3 * H),
      h0.reshape(n_graphs, H), gru_Whh_f.T, gru_bhh_f.reshape(1, 3 * H),
      gru_Whh_b.T, gru_bhh_b.reshape(1, 3 * H),
      W_o[:, :H].T, W_o[:, H:].T, b_o.reshape(1, H))
    return out
